# baseline (device time: 166791 ns/iter reference)
import functools

import jax
import jax.numpy as jnp
from jax import lax
from jax.experimental import pallas as pl
from jax.experimental.pallas import tpu as pltpu

N_DEV = 16


def kernel(dy, W):
    m, k = dy.shape
    d, k2 = W.shape
    assert k == k2
    chunk = m // N_DEV

    def body(dy_ref, w_ref, out_ref, comm_ref, send_sem, rs_sems, ag_sems):
        me = lax.axis_index("i")
        left = (me - 1) % N_DEV
        right = (me + 1) % N_DEV

        barrier_sem = pltpu.get_barrier_semaphore()
        for nbr in (left, right):
            pl.semaphore_signal(
                barrier_sem, inc=1,
                device_id=(nbr,), device_id_type=pl.DeviceIdType.MESH,
            )
        pl.semaphore_wait(barrier_sem, 2)

        out_ref[...] = lax.dot_general(
            dy_ref[...], w_ref[...],
            dimension_numbers=(((1,), (1,)), ((), ())),
            preferred_element_type=jnp.float32,
        )

        for s in range(N_DEV - 1):
            send_c = (me - s) % N_DEV
            recv_c = (me - s - 1) % N_DEV
            rdma = pltpu.make_async_remote_copy(
                src_ref=out_ref.at[pl.ds(send_c * chunk, chunk), :],
                dst_ref=comm_ref.at[s],
                send_sem=send_sem,
                recv_sem=rs_sems.at[s],
                device_id=(right,),
                device_id_type=pl.DeviceIdType.MESH,
            )
            rdma.start()
            rdma.wait()
            out_ref[pl.ds(recv_c * chunk, chunk), :] += comm_ref[s]

        for s in range(N_DEV - 1):
            send_c = (me + 1 - s) % N_DEV
            rdma = pltpu.make_async_remote_copy(
                src_ref=out_ref.at[pl.ds(send_c * chunk, chunk), :],
                dst_ref=out_ref.at[pl.ds(send_c * chunk, chunk), :],
                send_sem=send_sem,
                recv_sem=ag_sems.at[s],
                device_id=(right,),
                device_id_type=pl.DeviceIdType.MESH,
            )
            rdma.start()
            rdma.wait()

        @functools.partial(
            pl.run_scoped, exit_sem=pltpu.SemaphoreType.REGULAR
        )
        def _(exit_sem):
            for nbr in (left, right):
                pl.semaphore_signal(
                    exit_sem, inc=1,
                    device_id=(nbr,), device_id_type=pl.DeviceIdType.MESH,
                )
            pl.semaphore_wait(exit_sem, 2)

    return pl.pallas_call(
        body,
        out_shape=jax.ShapeDtypeStruct((m, d), jnp.float32),
        in_specs=[
            pl.BlockSpec(memory_space=pltpu.VMEM),
            pl.BlockSpec(memory_space=pltpu.VMEM),
        ],
        out_specs=pl.BlockSpec(memory_space=pltpu.VMEM),
        scratch_shapes=[
            pltpu.VMEM((N_DEV - 1, chunk, d), jnp.float32),
            pltpu.SemaphoreType.DMA,
            pltpu.SemaphoreType.DMA((N_DEV - 1,)),
            pltpu.SemaphoreType.DMA((N_DEV - 1,)),
        ],
        compiler_params=pltpu.CompilerParams(collective_id=0),
    )(dy, W)


# device time: 101196 ns/iter; 1.6482x vs baseline; 1.6482x over previous
import jax
import jax.numpy as jnp
from jax import lax
from jax.experimental import pallas as pl
from jax.experimental.pallas import tpu as pltpu

N_DEV = 16
P = 4
J = 4
RA = 128
RB = 32
BOT = 512


def kernel(dy, W):
    m, k = dy.shape
    d, _ = W.shape

    def body(dy_ref, w_ref, out_ref,
             abufR, abufL, bbufR, bbufL,
             send_sems, semAR, semAL, semBR, semBL):
        me = lax.axis_index("i")
        p = me // J
        j = me % J
        right_a = p * J + (j + 1) % J
        left_a = p * J + (j - 1) % J
        right_b = ((p + 1) % P) * J + j
        left_b = ((p - 1) % P) * J + j

        def xchg(src, dst, nbr, ss, rs):
            return pltpu.make_async_remote_copy(
                src_ref=src, dst_ref=dst, send_sem=ss, recv_sem=rs,
                device_id=(nbr,), device_id_type=pl.DeviceIdType.MESH,
            )

        out_ref[...] = lax.dot_general(
            dy_ref[...], w_ref[...],
            dimension_numbers=(((1,), (1,)), ((), ())),
            preferred_element_type=jnp.float32,
        )

        barrier_sem = pltpu.get_barrier_semaphore()
        for nbr in (right_a, left_a, right_b, left_b):
            pl.semaphore_signal(
                barrier_sem, inc=1,
                device_id=(nbr,), device_id_type=pl.DeviceIdType.MESH,
            )
        pl.semaphore_wait(barrier_sem, 4)

        for s in range(J - 1):
            cRs = (j - s) % J
            cRr = (j - s - 1) % J
            cLs = (j + s) % J
            cLr = (j + s + 1) % J
            r1 = xchg(out_ref.at[pl.ds(cRs * RA, RA), :], abufR.at[s],
                      right_a, send_sems.at[0], semAR.at[s])
            r2 = xchg(out_ref.at[pl.ds(BOT + cLs * RA, RA), :], abufL.at[s],
                      left_a, send_sems.at[1], semAL.at[s])
            r1.start()
            r2.start()
            r1.wait()
            r2.wait()
            out_ref[pl.ds(cRr * RA, RA), :] += abufR[s]
            out_ref[pl.ds(BOT + cLr * RA, RA), :] += abufL[s]

        top = ((j + 1) % J) * RA
        bot = BOT + ((j - 1) % J) * RA

        for s in range(P - 1):
            bRs = (p - s) % P
            bRr = (p - s - 1) % P
            bLs = (p + s) % P
            bLr = (p + s + 1) % P
            r1 = xchg(out_ref.at[pl.ds(top + bRs * RB, RB), :], bbufR.at[s],
                      right_b, send_sems.at[2], semBR.at[s])
            r2 = xchg(out_ref.at[pl.ds(bot + bLs * RB, RB), :], bbufL.at[s],
                      left_b, send_sems.at[3], semBL.at[s])
            r1.start()
            r2.start()
            r1.wait()
            r2.wait()
            out_ref[pl.ds(top + bRr * RB, RB), :] += bbufR[s]
            out_ref[pl.ds(bot + bLr * RB, RB), :] += bbufL[s]

        for s in range(P - 1):
            offR = top + ((p + 1 - s) % P) * RB
            offL = bot + ((p - 1 + s) % P) * RB
            r1 = xchg(out_ref.at[pl.ds(offR, RB), :],
                      out_ref.at[pl.ds(offR, RB), :],
                      right_b, send_sems.at[2], semBR.at[P - 1 + s])
            r2 = xchg(out_ref.at[pl.ds(offL, RB), :],
                      out_ref.at[pl.ds(offL, RB), :],
                      left_b, send_sems.at[3], semBL.at[P - 1 + s])
            r1.start()
            r2.start()
            r1.wait()
            r2.wait()

        for s in range(J - 1):
            offR = ((j + 1 - s) % J) * RA
            offL = BOT + ((j - 1 + s) % J) * RA
            r1 = xchg(out_ref.at[pl.ds(offR, RA), :],
                      out_ref.at[pl.ds(offR, RA), :],
                      right_a, send_sems.at[0], semAR.at[J - 1 + s])
            r2 = xchg(out_ref.at[pl.ds(offL, RA), :],
                      out_ref.at[pl.ds(offL, RA), :],
                      left_a, send_sems.at[1], semAL.at[J - 1 + s])
            r1.start()
            r2.start()
            r1.wait()
            r2.wait()

    return pl.pallas_call(
        body,
        out_shape=jax.ShapeDtypeStruct((m, d), jnp.float32),
        in_specs=[
            pl.BlockSpec(memory_space=pltpu.VMEM),
            pl.BlockSpec(memory_space=pltpu.VMEM),
        ],
        out_specs=pl.BlockSpec(memory_space=pltpu.VMEM),
        scratch_shapes=[
            pltpu.VMEM((J - 1, RA, d), jnp.float32),
            pltpu.VMEM((J - 1, RA, d), jnp.float32),
            pltpu.VMEM((P - 1, RB, d), jnp.float32),
            pltpu.VMEM((P - 1, RB, d), jnp.float32),
            pltpu.SemaphoreType.DMA((4,)),
            pltpu.SemaphoreType.DMA((2 * (J - 1),)),
            pltpu.SemaphoreType.DMA((2 * (J - 1),)),
            pltpu.SemaphoreType.DMA((2 * (P - 1),)),
            pltpu.SemaphoreType.DMA((2 * (P - 1),)),
        ],
        compiler_params=pltpu.CompilerParams(collective_id=0),
    )(dy, W)


# device time: 88028 ns/iter; 1.8947x vs baseline; 1.1496x over previous
import jax
import jax.numpy as jnp
from jax import lax
from jax.experimental import pallas as pl
from jax.experimental.pallas import tpu as pltpu

N_DEV = 16
P = 4
J = 4
RA = 128
RB = 32
BOT = 512
NSUB = 2


def kernel(dy, W):
    m, k = dy.shape
    d, _ = W.shape
    sa = RA // NSUB
    sb = RB // NSUB

    def body(dy_ref, w_ref, out_ref,
             abufR, abufL, bbufR, bbufL,
             ssAR, ssAL, ssBR, ssBL,
             semAR, semAL, semBR, semBL):
        me = lax.axis_index("i")
        p = me // J
        j = me % J
        nbr_a = (p * J + (j + 1) % J, p * J + (j - 1) % J)
        nbr_b = (((p + 1) % P) * J + j, ((p - 1) % P) * J + j)

        def gemm(off):
            out_ref[pl.ds(off, RA), :] = lax.dot_general(
                dy_ref[pl.ds(off, RA), :], w_ref[...],
                dimension_numbers=(((1,), (1,)), ((), ())),
                preferred_element_type=jnp.float32,
            )

        def rs_phase(nsteps, srows, send_off, recv_off, nbr,
                     bufs, ssems, rsems, hook):
            live = {}

            def start(dirn, s, h):
                off = send_off(dirn, s) + h * srows
                r = pltpu.make_async_remote_copy(
                    src_ref=out_ref.at[pl.ds(off, srows), :],
                    dst_ref=bufs[dirn].at[s, h],
                    send_sem=ssems[dirn].at[h],
                    recv_sem=rsems[dirn].at[s, h],
                    device_id=(nbr[dirn],),
                    device_id_type=pl.DeviceIdType.MESH,
                )
                r.start()
                live[(dirn, s, h)] = r

            for h in range(NSUB):
                start(0, 0, h)
                start(1, 0, h)
            hook(0)
            for s in range(nsteps):
                for h in range(NSUB):
                    for dirn in (0, 1):
                        r = live[(dirn, s, h)]
                        r.wait_recv()
                        off = recv_off(dirn, s) + h * srows
                        out_ref[pl.ds(off, srows), :] += bufs[dirn][s, h]
                        r.wait_send()
                        if s + 1 < nsteps:
                            start(dirn, s + 1, h)
                if s + 1 < nsteps:
                    hook(s + 1)

        def ag_phase(nsteps, srows, send_off, nbr, ssems, rsems, base):
            live = {}

            def start(dirn, s, h):
                off = send_off(dirn, s) + h * srows
                r = pltpu.make_async_remote_copy(
                    src_ref=out_ref.at[pl.ds(off, srows), :],
                    dst_ref=out_ref.at[pl.ds(off, srows), :],
                    send_sem=ssems[dirn].at[h],
                    recv_sem=rsems[dirn].at[base + s, h],
                    device_id=(nbr[dirn],),
                    device_id_type=pl.DeviceIdType.MESH,
                )
                r.start()
                live[(dirn, s, h)] = r

            for h in range(NSUB):
                start(0, 0, h)
                start(1, 0, h)
            for s in range(nsteps):
                for h in range(NSUB):
                    for dirn in (0, 1):
                        r = live[(dirn, s, h)]
                        r.wait_recv()
                        r.wait_send()
                        if s + 1 < nsteps:
                            start(dirn, s + 1, h)

        gemm(j * RA)
        gemm(BOT + j * RA)

        barrier_sem = pltpu.get_barrier_semaphore()
        for nbr in (*nbr_a, *nbr_b):
            pl.semaphore_signal(
                barrier_sem, inc=1,
                device_id=(nbr,), device_id_type=pl.DeviceIdType.MESH,
            )
        pl.semaphore_wait(barrier_sem, 4)

        rs_phase(
            J - 1, sa,
            lambda dirn, s: (((j - s) % J) * RA if dirn == 0
                             else BOT + ((j + s) % J) * RA),
            lambda dirn, s: (((j - s - 1) % J) * RA if dirn == 0
                             else BOT + ((j + s + 1) % J) * RA),
            nbr_a, (abufR, abufL), (ssAR, ssAL), (semAR, semAL),
            lambda s: (gemm(((j - s - 1) % J) * RA),
                       gemm(BOT + ((j + s + 1) % J) * RA)),
        )

        top = ((j + 1) % J) * RA
        bot = BOT + ((j - 1) % J) * RA

        rs_phase(
            P - 1, sb,
            lambda dirn, s: (top + ((p - s) % P) * RB if dirn == 0
                             else bot + ((p + s) % P) * RB),
            lambda dirn, s: (top + ((p - s - 1) % P) * RB if dirn == 0
                             else bot + ((p + s + 1) % P) * RB),
            nbr_b, (bbufR, bbufL), (ssBR, ssBL), (semBR, semBL),
            lambda s: None,
        )

        ag_phase(
            P - 1, sb,
            lambda dirn, s: (top + ((p + 1 - s) % P) * RB if dirn == 0
                             else bot + ((p - 1 + s) % P) * RB),
            nbr_b, (ssBR, ssBL), (semBR, semBL), P - 1,
        )

        ag_phase(
            J - 1, sa,
            lambda dirn, s: (((j + 1 - s) % J) * RA if dirn == 0
                             else BOT + ((j - 1 + s) % J) * RA),
            nbr_a, (ssAR, ssAL), (semAR, semAL), J - 1,
        )

    return pl.pallas_call(
        body,
        out_shape=jax.ShapeDtypeStruct((m, d), jnp.float32),
        in_specs=[
            pl.BlockSpec(memory_space=pltpu.VMEM),
            pl.BlockSpec(memory_space=pltpu.VMEM),
        ],
        out_specs=pl.BlockSpec(memory_space=pltpu.VMEM),
        scratch_shapes=[
            pltpu.VMEM((J - 1, NSUB, sa, d), jnp.float32),
            pltpu.VMEM((J - 1, NSUB, sa, d), jnp.float32),
            pltpu.VMEM((P - 1, NSUB, sb, d), jnp.float32),
            pltpu.VMEM((P - 1, NSUB, sb, d), jnp.float32),
            pltpu.SemaphoreType.DMA((NSUB,)),
            pltpu.SemaphoreType.DMA((NSUB,)),
            pltpu.SemaphoreType.DMA((NSUB,)),
            pltpu.SemaphoreType.DMA((NSUB,)),
            pltpu.SemaphoreType.DMA((2 * (J - 1), NSUB)),
            pltpu.SemaphoreType.DMA((2 * (J - 1), NSUB)),
            pltpu.SemaphoreType.DMA((2 * (P - 1), NSUB)),
            pltpu.SemaphoreType.DMA((2 * (P - 1), NSUB)),
        ],
        compiler_params=pltpu.CompilerParams(collective_id=0),
    )(dy, W)
